# baseline (speedup 1.0000x reference)
import functools

import jax
import jax.numpy as jnp
from jax import lax
from jax.experimental import pallas as pl
from jax.experimental.pallas import tpu as pltpu
from jax.experimental.pallas import tpu_sc as plsc

NUM_ROWS = 4096 * 26
NUM_COLS = 1000
NWORDS = NUM_ROWS * NUM_COLS
WIN = 128
NWIN = NWORDS // WIN
NC = 2
NS = 16
NW = NC * NS
ROWS_PER_W = NUM_ROWS // NW
LANES = 16
CROWS = 128
NCHUNK = ROWS_PER_W // CROWS

FROWS = 8000
NFILL = NWIN // FROWS
FDEPTH = 8
assert NWIN % FROWS == 0

_sc_mesh = plsc.VectorSubcoreMesh(core_axis_name="c", subcore_axis_name="s")
_tc_mesh = pltpu.create_tensorcore_mesh("tc", num_cores=1)


@functools.partial(
    pl.kernel,
    out_type=(),
    mesh=_tc_mesh,
    scratch_types=(
        pltpu.VMEM((FROWS, WIN), jnp.float32),
        pltpu.VMEM((FROWS, WIN), jnp.float32),
        pltpu.SemaphoreType.DMA,
    ),
)
def _tc_zero_fill(out_ref, zbuf0, zbuf1, fill_sem):
    zbuf0[...] = jnp.zeros_like(zbuf0)
    zbuf1[...] = jnp.zeros_like(zbuf1)

    def _dma(c, src):
        return pltpu.make_async_copy(
            src, out_ref.at[pl.ds(c * FROWS, FROWS), :], fill_sem)

    for d in range(FDEPTH):
        _dma(d, zbuf0 if d % 2 == 0 else zbuf1).start()

    def _steady(g, carry):
        _dma(2 * g, zbuf0).start()
        _dma(0, zbuf0).wait()
        _dma(2 * g + 1, zbuf1).start()
        _dma(0, zbuf1).wait()
        return carry

    lax.fori_loop(FDEPTH // 2, NFILL // 2, _steady, 0)

    def _drain(c, carry):
        _dma(0, zbuf0).wait()
        return carry

    lax.fori_loop(0, FDEPTH, _drain, 0)


@functools.partial(
    pl.kernel,
    out_type=(),
    mesh=_sc_mesh,
    scratch_types=(
        pltpu.VMEM((ROWS_PER_W + 16,), jnp.int32),
        pltpu.VMEM((NCHUNK, WIN), jnp.int32),
        pltpu.VMEM((CROWS, WIN), jnp.float32),
        pltpu.VMEM((CROWS, WIN), jnp.float32),
        pltpu.SemaphoreType.DMA,
        pltpu.SemaphoreType.DMA,
    ),
    compiler_params=pltpu.CompilerParams(needs_layout_passes=False),
)
def _sc_scatter_ones(x_hbm, out_ref, idxe, wv, bbuf0, bbuf1, sem0, sem1):
    wid = lax.axis_index("s") * NC + lax.axis_index("c")
    base_row = wid * ROWS_PER_W

    pltpu.sync_copy(x_hbm.at[pl.ds(base_row, ROWS_PER_W)],
                    idxe.at[pl.ds(8, ROWS_PER_W)])

    @pl.when(wid > 0)
    def _():
        pltpu.sync_copy(x_hbm.at[pl.ds(base_row - 8, 8)],
                        idxe.at[pl.ds(0, 8)])

    @pl.when(wid < NW - 1)
    def _():
        pltpu.sync_copy(x_hbm.at[pl.ds(base_row + ROWS_PER_W, 8)],
                        idxe.at[pl.ds(ROWS_PER_W + 8, 8)])

    zeros16 = jnp.zeros((LANES,), jnp.float32)
    ones16 = jnp.ones((LANES,), jnp.float32)
    iota16 = lax.iota(jnp.int32, LANES)

    def _wv(r, carry):
        for k in range(WIN // LANES):
            off = r * WIN + k * LANES
            grow = base_row + off + iota16
            p = grow * NUM_COLS + idxe[pl.ds(off + 8, LANES)]
            wv[r, pl.ds(k * LANES, LANES)] = lax.shift_right_logical(p, 7)
        return carry

    lax.fori_loop(0, NCHUNK, _wv, 0)

    def _bzero(i, carry):
        for k in range(WIN // LANES):
            bbuf0[i, pl.ds(k * LANES, LANES)] = zeros16
            bbuf1[i, pl.ds(k * LANES, LANES)] = zeros16
        return carry

    lax.fori_loop(0, CROWS, _bzero, 0)

    def _paint(c, bbuf, val16):
        for k in range(CROWS // LANES):
            off = c * CROWS + k * LANES
            lrow = k * LANES + iota16
            grow = base_row + off + iota16
            p = grow * NUM_COLS + idxe[pl.ds(off + 8, LANES)]
            w = lax.shift_right_logical(p, 7)
            o = lax.bitwise_and(p, WIN - 1)
            pp = (grow - 1) * NUM_COLS + idxe[pl.ds(off + 7, LANES)]
            pn = (grow + 1) * NUM_COLS + idxe[pl.ds(off + 9, LANES)]
            wp = lax.shift_right_logical(pp, 7)
            wn = lax.shift_right_logical(pn, 7)
            op = lax.bitwise_and(pp, WIN - 1)
            on = lax.bitwise_and(pn, WIN - 1)
            mp = (wp == w) & (grow > 0)
            mn = (wn == w) & (grow < NUM_ROWS - 1)
            plsc.store_scatter(bbuf, [lrow, o], val16)
            plsc.store_scatter(bbuf, [lrow, op], val16, mask=mp)
            plsc.store_scatter(bbuf, [lrow, on], val16, mask=mn)

    def _dma(c, bbuf, sem):
        return pltpu.make_async_copy(bbuf, out_ref.at[wv.at[c]], sem)

    _paint(0, bbuf0, ones16)
    _dma(0, bbuf0, sem0).start()
    _paint(1, bbuf1, ones16)
    _dma(1, bbuf1, sem1).start()

    def _chunk(g, carry):
        _dma(0, bbuf0, sem0).wait()
        _paint(2 * g - 2, bbuf0, zeros16)
        _paint(2 * g, bbuf0, ones16)
        _dma(2 * g, bbuf0, sem0).start()
        _dma(0, bbuf1, sem1).wait()
        _paint(2 * g - 1, bbuf1, zeros16)
        _paint(2 * g + 1, bbuf1, ones16)
        _dma(2 * g + 1, bbuf1, sem1).start()
        return carry

    lax.fori_loop(1, NCHUNK // 2, _chunk, 0)
    _dma(0, bbuf0, sem0).wait()
    _dma(0, bbuf1, sem1).wait()


def kernel(x):
    xf = x.reshape(-1).astype(jnp.int32)
    out_ref = jax.new_ref(pl.empty((NWIN, WIN), jnp.float32))
    _tc_zero_fill(out_ref)
    _sc_scatter_ones(xf, out_ref)
    return jax.freeze(out_ref).reshape(4096, 26, NUM_COLS)


# speedup vs baseline: 1.0149x; 1.0149x over previous
import functools

import jax
import jax.numpy as jnp
from jax import lax
from jax.experimental import pallas as pl
from jax.experimental.pallas import tpu as pltpu
from jax.experimental.pallas import tpu_sc as plsc

NUM_ROWS = 4096 * 26
NUM_COLS = 1000
NC = 2
NS = 16
NW = NC * NS
ROWS_PER_W = NUM_ROWS // NW
LANES = 16
CHUNK = LANES
NCHUNKS = ROWS_PER_W // CHUNK
NBUF = 4
BUF_WORDS = CHUNK * NUM_COLS

_mesh = plsc.VectorSubcoreMesh(core_axis_name="c", subcore_axis_name="s")


@functools.partial(
    pl.kernel,
    out_type=jax.ShapeDtypeStruct((NUM_ROWS * NUM_COLS,), jnp.float32),
    mesh=_mesh,
    scratch_types=(
        [pltpu.VMEM((ROWS_PER_W,), jnp.int32)]
        + [pltpu.VMEM((BUF_WORDS,), jnp.float32) for _ in range(NBUF)]
        + [pltpu.SemaphoreType.DMA for _ in range(NBUF)]
    ),
    compiler_params=pltpu.CompilerParams(needs_layout_passes=False),
)
def _one_hot_sc(x_hbm, out_hbm, idx_v, b0, b1, b2, b3, s0, s1, s2, s3):
    bufs = [b0, b1, b2, b3]
    sems = [s0, s1, s2, s3]
    wid = lax.axis_index("s") * NC + lax.axis_index("c")
    base_row = wid * ROWS_PER_W

    pltpu.sync_copy(x_hbm.at[pl.ds(base_row, ROWS_PER_W)], idx_v)

    zeros16 = jnp.zeros((LANES,), jnp.float32)
    ones16 = jnp.ones((LANES,), jnp.float32)
    row_off = lax.iota(jnp.int32, 16) * NUM_COLS

    def _zero(i, carry):
        for b in range(NBUF):
            bufs[b][pl.ds(i * LANES, LANES)] = zeros16
        return carry

    lax.fori_loop(0, BUF_WORDS // LANES, _zero, 0)

    def scatter_ones(b, c):
        idx = idx_v[pl.ds(c * CHUNK, CHUNK)]
        plsc.store_scatter(bufs[b], [row_off + idx], ones16)

    def dma(b, c):
        dst = out_hbm.at[pl.ds((base_row + c * CHUNK) * NUM_COLS, BUF_WORDS)]
        return pltpu.make_async_copy(bufs[b], dst, sems[b])

    for b in range(NBUF):
        scatter_ones(b, b)
        dma(b, b).start()

    def step(g, carry):
        for b in range(NBUF):
            c = g * NBUF + b
            dma(b, c).wait()
            old_idx = idx_v[pl.ds((c - NBUF) * CHUNK, CHUNK)]
            plsc.store_scatter(bufs[b], [row_off + old_idx], zeros16)
            scatter_ones(b, c)
            dma(b, c).start()
        return carry

    lax.fori_loop(1, NCHUNKS // NBUF, step, 0)

    for b in range(NBUF):
        dma(b, 0).wait()


def kernel(x):
    x = x.reshape(-1).astype(jnp.int32)
    out = _one_hot_sc(x)
    return out.reshape(4096, 26, NUM_COLS)
